# Initial kernel scaffold; baseline (speedup 1.0000x reference)
import jax
import jax.numpy as jnp
from jax.experimental import pallas as pl


def kernel(indices, weight):
    raise NotImplementedError("write your pallas kernel here")


# speedup vs baseline: 1.3014x; 1.3014x over previous
import functools

import jax
import jax.numpy as jnp
from jax import lax
from jax.experimental import pallas as pl
from jax.experimental.pallas import tpu as pltpu
from jax.experimental.pallas import tpu_sc as plsc

_BATCH = 16384
_FIELDS = 26
_D = 32
_B = _BATCH * _FIELDS
_NC = 2
_NS = 16
_NW = _NC * _NS
_BPW = _B // _NW
_CH = 1024
_NCHUNK = _BPW // _CH


def _sc_gather_relu(idx_flat, weight):
    mesh = plsc.VectorSubcoreMesh(core_axis_name="c", subcore_axis_name="s")

    @functools.partial(
        pl.kernel,
        mesh=mesh,
        out_type=jax.ShapeDtypeStruct((_B, _D), jnp.float32),
        compiler_params=pltpu.CompilerParams(use_tc_tiling_on_sc=False),
        scratch_types=[
            pltpu.VMEM((_BPW,), jnp.int32),
            pltpu.VMEM((_CH, _D), jnp.float32),
            pltpu.SemaphoreType.DMA,
        ],
    )
    def k(idx_hbm, table_hbm, out_hbm, idx_v, rows_v, sem):
        wid = lax.axis_index("s") * _NC + lax.axis_index("c")
        base = wid * _BPW
        pltpu.sync_copy(idx_hbm.at[pl.ds(base, _BPW)], idx_v)

        @pl.loop(0, _NCHUNK)
        def _(ci):
            off = ci * _CH
            pltpu.async_copy(
                table_hbm.at[idx_v.at[pl.ds(off, _CH)]], rows_v, sem
            ).wait()

            @pl.loop(0, _CH)
            def _(r):
                for c in range(0, _D, 16):
                    rows_v[r, pl.ds(c, 16)] = jnp.maximum(
                        rows_v[r, pl.ds(c, 16)], 0.0
                    )

            pltpu.sync_copy(rows_v, out_hbm.at[pl.ds(base + off, _CH)])

    return k(idx_flat, weight)


def kernel(indices, weight):
    idx_flat = indices.reshape(-1).astype(jnp.int32)
    out = _sc_gather_relu(idx_flat, weight)
    return out.reshape(_BATCH, _FIELDS, _D)


# speedup vs baseline: 2.2960x; 1.7642x over previous
import functools

import jax
import jax.numpy as jnp
from jax import lax
from jax.experimental import pallas as pl
from jax.experimental.pallas import tpu as pltpu
from jax.experimental.pallas import tpu_sc as plsc

_DATASET = 1000000
_BATCH = 16384
_FIELDS = 26
_D = 32
_B = _BATCH * _FIELDS
_NC = 2
_NS = 16
_NW = _NC * _NS
_BPW = _B // _NW
_CH = 1024
_NCHUNK = _BPW // _CH

_T1_W = 8192
_T1_Q = _T1_W // 4
_T1_STEPS = -(-_DATASET // _T1_W)
_T1_ROWS = _T1_STEPS * _T1_Q

_T2_U = _BATCH // 4


def _pack_table_body(wt_ref, out_ref):
    x = wt_ref[...]
    parts = [x[:, a * _T1_Q:(a + 1) * _T1_Q].T for a in range(4)]
    out_ref[...] = jnp.concatenate(parts, axis=1)


def _pack_table(wt):
    return pl.pallas_call(
        _pack_table_body,
        grid=(_T1_STEPS,),
        in_specs=[pl.BlockSpec((_D, _T1_W), lambda i: (0, i))],
        out_specs=pl.BlockSpec((_T1_Q, 128), lambda i: (i, 0)),
        out_shape=jax.ShapeDtypeStruct((_T1_ROWS, 128), jnp.float32),
        compiler_params=pltpu.CompilerParams(
            dimension_semantics=("parallel",),
        ),
    )(wt)


def _sc_gather(idx_flat, table):
    mesh = plsc.VectorSubcoreMesh(core_axis_name="c", subcore_axis_name="s")

    @functools.partial(
        pl.kernel,
        mesh=mesh,
        out_type=jax.ShapeDtypeStruct((_B, _D), jnp.float32),
        compiler_params=pltpu.CompilerParams(use_tc_tiling_on_sc=False),
        scratch_types=[
            pltpu.VMEM((_BPW,), jnp.int32),
            pltpu.VMEM((2, _CH, _D), jnp.float32),
            pltpu.SemaphoreType.DMA((2,)),
            pltpu.SemaphoreType.DMA((2,)),
        ],
    )
    def k(idx_hbm, table_hbm, out_hbm, idx_v, rows_v, gsem, wsem):
        wid = lax.axis_index("s") * _NC + lax.axis_index("c")
        base = wid * _BPW
        pltpu.sync_copy(idx_hbm.at[pl.ds(base, _BPW)], idx_v)

        def gather(ci, buf):
            return pltpu.make_async_copy(
                table_hbm.at[idx_v.at[pl.ds(ci * _CH, _CH)]],
                rows_v.at[buf],
                gsem.at[buf],
            )

        def write(ci, buf):
            return pltpu.make_async_copy(
                rows_v.at[buf],
                out_hbm.at[pl.ds(base + ci * _CH, _CH)],
                wsem.at[buf],
            )

        gather(0, 0).start()
        gather(1, 1).start()

        @pl.loop(0, _NCHUNK - 2)
        def _(ci):
            buf = lax.rem(ci, 2)
            gather(ci, buf).wait()
            write(ci, buf).start()
            write(ci, buf).wait()
            gather(ci + 2, buf).start()

        @pl.loop(_NCHUNK - 2, _NCHUNK)
        def _(ci):
            buf = lax.rem(ci, 2)
            gather(ci, buf).wait()
            write(ci, buf).start()
            write(ci, buf).wait()

    return k(idx_flat, table)


def _unpack_relu_body(g_ref, o_ref):
    x = g_ref[...]
    parts = [
        jnp.maximum(x[:, a * _D:(a + 1) * _D].T, 0.0) for a in range(4)
    ]
    o_ref[0] = jnp.concatenate(parts, axis=1)


def _unpack_relu(g128):
    return pl.pallas_call(
        _unpack_relu_body,
        grid=(_FIELDS,),
        in_specs=[pl.BlockSpec((_T2_U, 128), lambda f: (f, 0))],
        out_specs=pl.BlockSpec((1, _D, _BATCH), lambda f: (f, 0, 0)),
        out_shape=jax.ShapeDtypeStruct((_FIELDS, _D, _BATCH), jnp.float32),
        compiler_params=pltpu.CompilerParams(
            dimension_semantics=("parallel",),
        ),
    )(g128)


def kernel(indices, weight):
    wt = weight.T
    w2 = _pack_table(wt)
    table = w2.reshape(_T1_ROWS * 4, _D)

    ti = indices.T.astype(jnp.int32)
    itp = ti.reshape(_FIELDS, 4, _T2_U).transpose(0, 2, 1).reshape(-1)
    vb = itp // _T1_W
    u = itp % _T1_W
    a = u // _T1_Q
    j = u % _T1_Q
    fidx = 4 * (vb * _T1_Q + j) + a

    out_lin = _sc_gather(fidx, table)
    g128 = out_lin.reshape(_B * _D // 128, 128)
    o_phys = _unpack_relu(g128)
    return o_phys.transpose(2, 0, 1)


# speedup vs baseline: 3.5452x; 1.5440x over previous
import functools

import jax
import jax.numpy as jnp
from jax import lax
from jax.experimental import pallas as pl
from jax.experimental.pallas import tpu as pltpu
from jax.experimental.pallas import tpu_sc as plsc

_DATASET = 1000000
_BATCH = 16384
_FIELDS = 26
_D = 32
_B = _BATCH * _FIELDS
_NC = 2
_NS = 16
_NW = _NC * _NS
_BPW = _B // _NW
_CH = 1024
_NCHUNK = _BPW // _CH

_T1_W = 8192
_T1_Q = _T1_W // 4
_T1_STEPS = -(-_DATASET // _T1_W)
_T1_ROWS = _T1_STEPS * _T1_Q

_T2_U = _BATCH // 4


def _pack_table_body(wt_ref, out_ref):
    x = wt_ref[...]
    z = jnp.concatenate(
        [x[:, a * _T1_Q:(a + 1) * _T1_Q] for a in range(4)], axis=0
    )
    out_ref[...] = z.T


def _pack_table(wt):
    return pl.pallas_call(
        _pack_table_body,
        grid=(_T1_STEPS,),
        in_specs=[pl.BlockSpec((_D, _T1_W), lambda i: (0, i))],
        out_specs=pl.BlockSpec((_T1_Q, 128), lambda i: (i, 0)),
        out_shape=jax.ShapeDtypeStruct((_T1_ROWS, 128), jnp.float32),
        compiler_params=pltpu.CompilerParams(
            dimension_semantics=("parallel",),
        ),
    )(wt)


def _sc_gather(idx_flat, table):
    mesh = plsc.VectorSubcoreMesh(core_axis_name="c", subcore_axis_name="s")

    @functools.partial(
        pl.kernel,
        mesh=mesh,
        out_type=jax.ShapeDtypeStruct((_B, _D), jnp.float32),
        compiler_params=pltpu.CompilerParams(use_tc_tiling_on_sc=False),
        scratch_types=[
            pltpu.VMEM((_BPW,), jnp.int32),
            pltpu.VMEM((2, _CH, _D), jnp.float32),
            pltpu.SemaphoreType.DMA((2,)),
            pltpu.SemaphoreType.DMA((2,)),
        ],
    )
    def k(idx_hbm, table_hbm, out_hbm, idx_v, rows_v, gsem, wsem):
        wid = lax.axis_index("s") * _NC + lax.axis_index("c")
        base = wid * _BPW
        pltpu.sync_copy(idx_hbm.at[pl.ds(base, _BPW)], idx_v)

        def gather(ci, buf):
            return pltpu.make_async_copy(
                table_hbm.at[idx_v.at[pl.ds(ci * _CH, _CH)]],
                rows_v.at[buf],
                gsem.at[buf],
            )

        def write(ci, buf):
            return pltpu.make_async_copy(
                rows_v.at[buf],
                out_hbm.at[pl.ds(base + ci * _CH, _CH)],
                wsem.at[buf],
            )

        gather(0, 0).start()
        gather(1, 1).start()

        @pl.loop(0, _NCHUNK - 2)
        def _(ci):
            buf = lax.rem(ci, 2)
            gather(ci, buf).wait()
            write(ci, buf).start()
            write(ci, buf).wait()
            gather(ci + 2, buf).start()

        @pl.loop(_NCHUNK - 2, _NCHUNK)
        def _(ci):
            buf = lax.rem(ci, 2)
            gather(ci, buf).wait()
            write(ci, buf).start()
            write(ci, buf).wait()

    return k(idx_flat, table)


def _unpack_relu_body(g_ref, o_ref):
    x = g_ref[...]
    xt = x.T
    parts = [xt[a * _D:(a + 1) * _D, :] for a in range(4)]
    o_ref[0] = jnp.maximum(jnp.concatenate(parts, axis=1), 0.0)


def _unpack_relu(g128):
    return pl.pallas_call(
        _unpack_relu_body,
        grid=(_FIELDS,),
        in_specs=[pl.BlockSpec((_T2_U, 128), lambda f: (f, 0))],
        out_specs=pl.BlockSpec((1, _D, _BATCH), lambda f: (f, 0, 0)),
        out_shape=jax.ShapeDtypeStruct((_FIELDS, _D, _BATCH), jnp.float32),
        compiler_params=pltpu.CompilerParams(
            dimension_semantics=("parallel",),
        ),
    )(g128)


def kernel(indices, weight):
    wt = weight.T
    w2 = _pack_table(wt)
    table = w2.reshape(_T1_ROWS * 4, _D)

    ti = indices.T.astype(jnp.int32)
    itp = ti.reshape(_FIELDS, 4, _T2_U).transpose(0, 2, 1).reshape(-1)
    vb = itp // _T1_W
    u = itp % _T1_W
    a = u // _T1_Q
    j = u % _T1_Q
    fidx = 4 * (vb * _T1_Q + j) + a

    out_lin = _sc_gather(fidx, table)
    g128 = out_lin.reshape(_B * _D // 128, 128)
    o_phys = _unpack_relu(g128)
    return o_phys.transpose(2, 0, 1)


# speedup vs baseline: 4.6073x; 1.2996x over previous
import functools

import jax
import jax.numpy as jnp
from jax import lax
from jax.experimental import pallas as pl
from jax.experimental.pallas import tpu as pltpu
from jax.experimental.pallas import tpu_sc as plsc

_DATASET = 1000000
_BATCH = 16384
_FIELDS = 26
_D = 32
_B = _BATCH * _FIELDS
_NC = 2
_NS = 16
_NW = _NC * _NS
_BPW = _B // _NW
_CH = 1024
_NCHUNK = _BPW // _CH

_T1_W = 8192
_T1_Q = _T1_W // 4
_T1_STEPS = -(-_DATASET // _T1_W)
_T1_ROWS = _T1_STEPS * _T1_Q

_T2_U = _BATCH // 4


def _pack_table_body(wt_ref, out_ref):
    x = wt_ref[...]
    z = jnp.concatenate(
        [x[:, a * _T1_Q:(a + 1) * _T1_Q] for a in range(4)], axis=0
    )
    out_ref[...] = z.T


def _pack_table(wt):
    return pl.pallas_call(
        _pack_table_body,
        grid=(_T1_STEPS,),
        in_specs=[pl.BlockSpec((_D, _T1_W), lambda i: (0, i))],
        out_specs=pl.BlockSpec((_T1_Q, 128), lambda i: (i, 0)),
        out_shape=jax.ShapeDtypeStruct((_T1_ROWS, 128), jnp.float32),
        compiler_params=pltpu.CompilerParams(
            dimension_semantics=("parallel",),
        ),
    )(wt)


def _sc_gather(idx_flat, table):
    mesh = plsc.VectorSubcoreMesh(core_axis_name="c", subcore_axis_name="s")

    @functools.partial(
        pl.kernel,
        mesh=mesh,
        out_type=jax.ShapeDtypeStruct((_B // 4, 128), jnp.float32),
        compiler_params=pltpu.CompilerParams(use_tc_tiling_on_sc=False),
        scratch_types=[
            pltpu.VMEM((_BPW,), jnp.int32),
            pltpu.VMEM((2, _CH, _D), jnp.float32),
            pltpu.SemaphoreType.DMA((2,)),
            pltpu.SemaphoreType.DMA((2,)),
        ],
    )
    def k(idx_hbm, table_hbm, out_hbm, idx_v, rows_v, gsem, wsem):
        wid = lax.axis_index("s") * _NC + lax.axis_index("c")
        base = wid * _BPW
        pltpu.sync_copy(idx_hbm.at[pl.ds(base, _BPW)], idx_v)

        def gather(ci, buf):
            return pltpu.make_async_copy(
                table_hbm.at[idx_v.at[pl.ds(ci * _CH, _CH)]],
                rows_v.at[buf],
                gsem.at[buf],
            )

        def write(ci, buf):
            g0 = base + ci * _CH
            p = g0 // (_BATCH // 4)
            u0 = lax.rem(g0, _BATCH // 4)
            c0 = (p // 4) * (_BATCH // 4) + u0
            a = lax.rem(p, 4)
            return pltpu.make_async_copy(
                rows_v.at[buf],
                out_hbm.at[pl.ds(c0, _CH), pl.ds(a * _D, _D)],
                wsem.at[buf],
            )

        gather(0, 0).start()
        gather(1, 1).start()

        @pl.loop(0, _NCHUNK - 2)
        def _(ci):
            buf = lax.rem(ci, 2)
            gather(ci, buf).wait()
            write(ci, buf).start()
            write(ci, buf).wait()
            gather(ci + 2, buf).start()

        @pl.loop(_NCHUNK - 2, _NCHUNK)
        def _(ci):
            buf = lax.rem(ci, 2)
            gather(ci, buf).wait()
            write(ci, buf).start()
            write(ci, buf).wait()

    return k(idx_flat, table)


def _unpack_relu_body(g_ref, o_ref):
    x = g_ref[...]
    xt = x.T
    parts = [xt[a * _D:(a + 1) * _D, :] for a in range(4)]
    o_ref[0] = jnp.maximum(jnp.concatenate(parts, axis=1), 0.0)


def _unpack_relu(g128):
    return pl.pallas_call(
        _unpack_relu_body,
        grid=(_FIELDS,),
        in_specs=[pl.BlockSpec((_T2_U, 128), lambda f: (f, 0))],
        out_specs=pl.BlockSpec((1, _D, _BATCH), lambda f: (f, 0, 0)),
        out_shape=jax.ShapeDtypeStruct((_FIELDS, _D, _BATCH), jnp.float32),
        compiler_params=pltpu.CompilerParams(
            dimension_semantics=("parallel",),
        ),
    )(g128)


def kernel(indices, weight):
    wt = weight.T
    w2 = _pack_table(wt)
    table = w2.reshape(_T1_ROWS * 4, _D)

    it = indices.T.reshape(-1).astype(jnp.int32)
    vb = it // _T1_W
    u = it % _T1_W
    a = u // _T1_Q
    j = u % _T1_Q
    fidx = 4 * (vb * _T1_Q + j) + a

    g128 = _sc_gather(fidx, table)
    o_phys = _unpack_relu(g128)
    return o_phys.transpose(2, 0, 1)


# speedup vs baseline: 5.8628x; 1.2725x over previous
import functools

import jax
import jax.numpy as jnp
from jax import lax
from jax.experimental import pallas as pl
from jax.experimental.pallas import tpu as pltpu
from jax.experimental.pallas import tpu_sc as plsc

_DATASET = 1000000
_BATCH = 16384
_FIELDS = 26
_D = 32
_B = _BATCH * _FIELDS
_NC = 2
_NS = 16
_NW = _NC * _NS
_BPW = _B // _NW
_CH = 512
_NCHUNK = _BPW // _CH
_NBUF = 4

_T1_W = 32768
_T1_Q = _T1_W // 4
_T1_STEPS = -(-_DATASET // _T1_W)
_T1_ROWS = _T1_STEPS * _T1_Q

_T2_U = _BATCH // 4


def _pack_table_body(wt_ref, out_ref):
    x = wt_ref[...]
    z = jnp.concatenate(
        [x[:, a * _T1_Q:(a + 1) * _T1_Q] for a in range(4)], axis=0
    )
    out_ref[...] = z.T


def _pack_table(wt):
    return pl.pallas_call(
        _pack_table_body,
        grid=(_T1_STEPS,),
        in_specs=[pl.BlockSpec((_D, _T1_W), lambda i: (0, i))],
        out_specs=pl.BlockSpec((_T1_Q, 128), lambda i: (i, 0)),
        out_shape=jax.ShapeDtypeStruct((_T1_ROWS, 128), jnp.float32),
        compiler_params=pltpu.CompilerParams(
            dimension_semantics=("parallel",),
        ),
    )(wt)


def _sc_gather(idx_flat, table):
    mesh = plsc.VectorSubcoreMesh(core_axis_name="c", subcore_axis_name="s")

    @functools.partial(
        pl.kernel,
        mesh=mesh,
        out_type=jax.ShapeDtypeStruct((_B // 4, 128), jnp.float32),
        compiler_params=pltpu.CompilerParams(use_tc_tiling_on_sc=False),
        scratch_types=[
            pltpu.VMEM((_BPW,), jnp.int32),
            pltpu.VMEM((_NBUF, _CH, _D), jnp.float32),
            pltpu.SemaphoreType.DMA((_NBUF,)),
            pltpu.SemaphoreType.DMA((_NBUF,)),
        ],
    )
    def k(idx_hbm, table_hbm, out_hbm, idx_v, rows_v, gsem, wsem):
        wid = lax.axis_index("s") * _NC + lax.axis_index("c")
        base = wid * _BPW
        pltpu.sync_copy(idx_hbm.at[pl.ds(base, _BPW)], idx_v)

        def gather(ci, buf):
            return pltpu.make_async_copy(
                table_hbm.at[idx_v.at[pl.ds(ci * _CH, _CH)]],
                rows_v.at[buf],
                gsem.at[buf],
            )

        def write(ci, buf):
            g0 = base + ci * _CH
            p = g0 // (_BATCH // 4)
            u0 = lax.rem(g0, _BATCH // 4)
            c0 = (p // 4) * (_BATCH // 4) + u0
            a = lax.rem(p, 4)
            return pltpu.make_async_copy(
                rows_v.at[buf],
                out_hbm.at[pl.ds(c0, _CH), pl.ds(a * _D, _D)],
                wsem.at[buf],
            )

        for k in range(_NBUF):
            gather(k, k).start()

        @pl.loop(0, _NCHUNK)
        def _(ci):
            buf = lax.rem(ci, _NBUF)
            prv = lax.rem(ci + _NBUF - 1, _NBUF)

            @pl.when(ci >= 1)
            def _():
                write(ci - 1, prv).wait()

            @pl.when(jnp.logical_and(ci >= 1, ci + _NBUF - 1 < _NCHUNK))
            def _():
                gather(ci + _NBUF - 1, prv).start()

            gather(ci, buf).wait()
            write(ci, buf).start()

        write(_NCHUNK - 1, lax.rem(_NCHUNK - 1, _NBUF)).wait()

    return k(idx_flat, table)


def _unpack_relu_body(g_ref, o_ref):
    x = g_ref[...]
    xt = x.T
    parts = [xt[a * _D:(a + 1) * _D, :] for a in range(4)]
    o_ref[0] = jnp.maximum(jnp.concatenate(parts, axis=1), 0.0)


def _unpack_relu(g128):
    return pl.pallas_call(
        _unpack_relu_body,
        grid=(_FIELDS,),
        in_specs=[pl.BlockSpec((_T2_U, 128), lambda f: (f, 0))],
        out_specs=pl.BlockSpec((1, _D, _BATCH), lambda f: (f, 0, 0)),
        out_shape=jax.ShapeDtypeStruct((_FIELDS, _D, _BATCH), jnp.float32),
        compiler_params=pltpu.CompilerParams(
            dimension_semantics=("parallel",),
        ),
    )(g128)


def kernel(indices, weight):
    wt = weight.T
    w2 = _pack_table(wt)
    table = w2.reshape(_T1_ROWS * 4, _D)

    it = indices.T.reshape(-1).astype(jnp.int32)
    vb = it // _T1_W
    u = it % _T1_W
    a = u // _T1_Q
    j = u % _T1_Q
    fidx = 4 * (vb * _T1_Q + j) + a

    g128 = _sc_gather(fidx, table)
    o_phys = _unpack_relu(g128)
    return o_phys.transpose(2, 0, 1)


# speedup vs baseline: 5.8785x; 1.0027x over previous
import functools

import jax
import jax.numpy as jnp
from jax import lax
from jax.experimental import pallas as pl
from jax.experimental.pallas import tpu as pltpu
from jax.experimental.pallas import tpu_sc as plsc

_DATASET = 1000000
_BATCH = 16384
_FIELDS = 26
_D = 32
_B = _BATCH * _FIELDS
_NC = 2
_NS = 16
_NW = _NC * _NS
_BPW = _B // _NW
_CH = 512
_NCHUNK = _BPW // _CH
_NBUF = 4

_T1_W = 65536
_T1_Q = _T1_W // 4
_T1_STEPS = -(-_DATASET // _T1_W)
_T1_ROWS = _T1_STEPS * _T1_Q

_T2_U = _BATCH // 4


def _pack_table_body(wt_ref, out_ref):
    x = wt_ref[...]
    z = jnp.concatenate(
        [x[:, a * _T1_Q:(a + 1) * _T1_Q] for a in range(4)], axis=0
    )
    out_ref[...] = z.T


def _pack_table(wt):
    return pl.pallas_call(
        _pack_table_body,
        grid=(_T1_STEPS,),
        in_specs=[pl.BlockSpec((_D, _T1_W), lambda i: (0, i))],
        out_specs=pl.BlockSpec((_T1_Q, 128), lambda i: (i, 0)),
        out_shape=jax.ShapeDtypeStruct((_T1_ROWS, 128), jnp.float32),
        compiler_params=pltpu.CompilerParams(
            dimension_semantics=("parallel",),
        ),
    )(wt)


def _sc_gather(idx_flat, table):
    mesh = plsc.VectorSubcoreMesh(core_axis_name="c", subcore_axis_name="s")

    @functools.partial(
        pl.kernel,
        mesh=mesh,
        out_type=jax.ShapeDtypeStruct((_B // 4, 128), jnp.float32),
        compiler_params=pltpu.CompilerParams(use_tc_tiling_on_sc=False),
        scratch_types=[
            pltpu.VMEM((_BPW,), jnp.int32),
            pltpu.VMEM((_NBUF, _CH, _D), jnp.float32),
            pltpu.SemaphoreType.DMA((_NBUF,)),
            pltpu.SemaphoreType.DMA((_NBUF,)),
        ],
    )
    def k(idx_hbm, table_hbm, out_hbm, idx_v, rows_v, gsem, wsem):
        wid = lax.axis_index("s") * _NC + lax.axis_index("c")
        base = wid * _BPW
        pltpu.sync_copy(idx_hbm.at[pl.ds(base, _BPW)], idx_v)

        def gather(ci, buf):
            return pltpu.make_async_copy(
                table_hbm.at[idx_v.at[pl.ds(ci * _CH, _CH)]],
                rows_v.at[buf],
                gsem.at[buf],
            )

        def write(ci, buf):
            g0 = base + ci * _CH
            p = g0 // (_BATCH // 4)
            u0 = lax.rem(g0, _BATCH // 4)
            c0 = (p // 4) * (_BATCH // 4) + u0
            a = lax.rem(p, 4)
            return pltpu.make_async_copy(
                rows_v.at[buf],
                out_hbm.at[pl.ds(c0, _CH), pl.ds(a * _D, _D)],
                wsem.at[buf],
            )

        for k in range(_NBUF):
            gather(k, k).start()

        @pl.loop(0, _NCHUNK)
        def _(ci):
            buf = lax.rem(ci, _NBUF)
            prv = lax.rem(ci + _NBUF - 1, _NBUF)

            @pl.when(ci >= 1)
            def _():
                write(ci - 1, prv).wait()

            @pl.when(jnp.logical_and(ci >= 1, ci + _NBUF - 1 < _NCHUNK))
            def _():
                gather(ci + _NBUF - 1, prv).start()

            gather(ci, buf).wait()
            write(ci, buf).start()

        write(_NCHUNK - 1, lax.rem(_NCHUNK - 1, _NBUF)).wait()

    return k(idx_flat, table)


def _unpack_relu_body(g_ref, o_ref):
    x = g_ref[...]
    xt = x.T
    parts = [xt[a * _D:(a + 1) * _D, :] for a in range(4)]
    o_ref[0] = jnp.maximum(jnp.concatenate(parts, axis=1), 0.0)


def _unpack_relu(g128):
    return pl.pallas_call(
        _unpack_relu_body,
        grid=(_FIELDS,),
        in_specs=[pl.BlockSpec((_T2_U, 128), lambda f: (f, 0))],
        out_specs=pl.BlockSpec((1, _D, _BATCH), lambda f: (f, 0, 0)),
        out_shape=jax.ShapeDtypeStruct((_FIELDS, _D, _BATCH), jnp.float32),
        compiler_params=pltpu.CompilerParams(
            dimension_semantics=("parallel",),
        ),
    )(g128)


def kernel(indices, weight):
    wt = weight.T
    w2 = _pack_table(wt)
    table = w2.reshape(_T1_ROWS * 4, _D)

    it = indices.T.reshape(-1).astype(jnp.int32)
    vb = it // _T1_W
    u = it % _T1_W
    a = u // _T1_Q
    j = u % _T1_Q
    fidx = 4 * (vb * _T1_Q + j) + a

    g128 = _sc_gather(fidx, table)
    o_phys = _unpack_relu(g128)
    return o_phys.transpose(2, 0, 1)


# speedup vs baseline: 6.0922x; 1.0364x over previous
import functools

import jax
import jax.numpy as jnp
from jax import lax
from jax.experimental import pallas as pl
from jax.experimental.pallas import tpu as pltpu
from jax.experimental.pallas import tpu_sc as plsc

_DATASET = 1000000
_BATCH = 16384
_FIELDS = 26
_D = 32
_B = _BATCH * _FIELDS
_NC = 2
_NS = 16
_NW = _NC * _NS
_BPW = _B // _NW
_CH = 512
_NCHUNK = _BPW // _CH
_NBUF = 4

_T1_W = 65536
_T1_Q = _T1_W // 4
_T1_STEPS = -(-_DATASET // _T1_W)
_T1_ROWS = _T1_STEPS * _T1_Q

_T2_U = _BATCH // 4


def _pack_table_body(wt_ref, out_ref):
    x = wt_ref[...]
    z = jnp.concatenate(
        [x[:, a * _T1_Q:(a + 1) * _T1_Q] for a in range(4)], axis=0
    )
    out_ref[...] = z.T


def _pack_table(wt):
    return pl.pallas_call(
        _pack_table_body,
        grid=(_T1_STEPS,),
        in_specs=[pl.BlockSpec((_D, _T1_W), lambda i: (0, i))],
        out_specs=pl.BlockSpec((_T1_Q, 128), lambda i: (i, 0)),
        out_shape=jax.ShapeDtypeStruct((_T1_ROWS, 128), jnp.float32),
        compiler_params=pltpu.CompilerParams(
            dimension_semantics=("parallel",),
        ),
    )(wt)


def _sc_gather(idx_flat, table):
    mesh = plsc.VectorSubcoreMesh(core_axis_name="c", subcore_axis_name="s")

    @functools.partial(
        pl.kernel,
        mesh=mesh,
        out_type=jax.ShapeDtypeStruct((_B // 4, 128), jnp.float32),
        compiler_params=pltpu.CompilerParams(use_tc_tiling_on_sc=False),
        scratch_types=[
            pltpu.VMEM((_BPW,), jnp.int32),
            pltpu.VMEM((_NBUF, _CH, _D), jnp.float32),
            pltpu.SemaphoreType.DMA((_NBUF,)),
            pltpu.SemaphoreType.DMA((_NBUF,)),
        ],
    )
    def k(idx_hbm, table_hbm, out_hbm, idx_v, rows_v, gsem, wsem):
        wid = lax.axis_index("s") * _NC + lax.axis_index("c")
        base = wid * _BPW
        pltpu.sync_copy(idx_hbm.at[pl.ds(base, _BPW)], idx_v)

        def gather(ci, buf):
            return pltpu.make_async_copy(
                table_hbm.at[idx_v.at[pl.ds(ci * _CH, _CH)]],
                rows_v.at[buf],
                gsem.at[buf],
            )

        def write(ci, buf):
            g0 = base + ci * _CH
            p = g0 // (_BATCH // 4)
            u0 = lax.rem(g0, _BATCH // 4)
            c0 = (p // 4) * (_BATCH // 4) + u0
            a = lax.rem(p, 4)
            return pltpu.make_async_copy(
                rows_v.at[buf],
                out_hbm.at[pl.ds(c0, _CH), pl.ds(a * _D, _D)],
                wsem.at[buf],
            )

        for k in range(_NBUF):
            gather(k, k).start()

        @pl.loop(0, _NCHUNK)
        def _(ci):
            buf = lax.rem(ci, _NBUF)
            prv = lax.rem(ci + _NBUF - 1, _NBUF)

            @pl.when(ci >= 1)
            def _():
                write(ci - 1, prv).wait()

            @pl.when(jnp.logical_and(ci >= 1, ci + _NBUF - 1 < _NCHUNK))
            def _():
                gather(ci + _NBUF - 1, prv).start()

            gather(ci, buf).wait()
            write(ci, buf).start()

        write(_NCHUNK - 1, lax.rem(_NCHUNK - 1, _NBUF)).wait()

    return k(idx_flat, table)


_T2_F = 2


def _unpack_relu_body(g_ref, o_ref):
    x = g_ref[...]
    xt = x.T
    for i in range(_T2_F):
        parts = [
            xt[a * _D:(a + 1) * _D, i * _T2_U:(i + 1) * _T2_U]
            for a in range(4)
        ]
        o_ref[i] = jnp.maximum(jnp.concatenate(parts, axis=1), 0.0)


def _unpack_relu(g128):
    return pl.pallas_call(
        _unpack_relu_body,
        grid=(_FIELDS // _T2_F,),
        in_specs=[pl.BlockSpec((_T2_F * _T2_U, 128), lambda f: (f, 0))],
        out_specs=pl.BlockSpec((_T2_F, _D, _BATCH), lambda f: (f, 0, 0)),
        out_shape=jax.ShapeDtypeStruct((_FIELDS, _D, _BATCH), jnp.float32),
        compiler_params=pltpu.CompilerParams(
            dimension_semantics=("parallel",),
        ),
    )(g128)


def kernel(indices, weight):
    wt = weight.T
    w2 = _pack_table(wt)
    table = w2.reshape(_T1_ROWS * 4, _D)

    it = indices.T.reshape(-1).astype(jnp.int32)
    vb = it // _T1_W
    u = it % _T1_W
    a = u // _T1_Q
    j = u % _T1_Q
    fidx = 4 * (vb * _T1_Q + j) + a

    g128 = _sc_gather(fidx, table)
    o_phys = _unpack_relu(g128)
    return o_phys.transpose(2, 0, 1)


# speedup vs baseline: 6.1093x; 1.0028x over previous
import functools

import jax
import jax.numpy as jnp
from jax import lax
from jax.experimental import pallas as pl
from jax.experimental.pallas import tpu as pltpu
from jax.experimental.pallas import tpu_sc as plsc

_DATASET = 1000000
_BATCH = 16384
_FIELDS = 26
_D = 32
_B = _BATCH * _FIELDS
_NC = 2
_NS = 16
_NW = _NC * _NS
_BPW = _B // _NW
_CH = 1024
_NCHUNK = _BPW // _CH
_NBUF = 3

_T1_W = 65536
_T1_Q = _T1_W // 4
_T1_STEPS = -(-_DATASET // _T1_W)
_T1_ROWS = _T1_STEPS * _T1_Q

_T2_U = _BATCH // 4


def _pack_table_body(wt_ref, out_ref):
    x = wt_ref[...]
    z = jnp.concatenate(
        [x[:, a * _T1_Q:(a + 1) * _T1_Q] for a in range(4)], axis=0
    )
    out_ref[...] = z.T


def _pack_table(wt):
    return pl.pallas_call(
        _pack_table_body,
        grid=(_T1_STEPS,),
        in_specs=[pl.BlockSpec((_D, _T1_W), lambda i: (0, i))],
        out_specs=pl.BlockSpec((_T1_Q, 128), lambda i: (i, 0)),
        out_shape=jax.ShapeDtypeStruct((_T1_ROWS, 128), jnp.float32),
        compiler_params=pltpu.CompilerParams(
            dimension_semantics=("parallel",),
        ),
    )(wt)


def _sc_gather(idx_flat, table):
    mesh = plsc.VectorSubcoreMesh(core_axis_name="c", subcore_axis_name="s")

    @functools.partial(
        pl.kernel,
        mesh=mesh,
        out_type=jax.ShapeDtypeStruct((_B // 4, 128), jnp.float32),
        compiler_params=pltpu.CompilerParams(use_tc_tiling_on_sc=False),
        scratch_types=[
            pltpu.VMEM((_BPW,), jnp.int32),
            pltpu.VMEM((_NBUF, _CH, _D), jnp.float32),
            pltpu.SemaphoreType.DMA((_NBUF,)),
            pltpu.SemaphoreType.DMA((_NBUF,)),
        ],
    )
    def k(idx_hbm, table_hbm, out_hbm, idx_v, rows_v, gsem, wsem):
        wid = lax.axis_index("s") * _NC + lax.axis_index("c")
        base = wid * _BPW
        pltpu.sync_copy(idx_hbm.at[pl.ds(base, _BPW)], idx_v)

        def gather(ci, buf):
            return pltpu.make_async_copy(
                table_hbm.at[idx_v.at[pl.ds(ci * _CH, _CH)]],
                rows_v.at[buf],
                gsem.at[buf],
            )

        def write(ci, buf):
            g0 = base + ci * _CH
            p = g0 // (_BATCH // 4)
            u0 = lax.rem(g0, _BATCH // 4)
            c0 = (p // 4) * (_BATCH // 4) + u0
            a = lax.rem(p, 4)
            return pltpu.make_async_copy(
                rows_v.at[buf],
                out_hbm.at[pl.ds(c0, _CH), pl.ds(a * _D, _D)],
                wsem.at[buf],
            )

        for k in range(_NBUF):
            gather(k, k).start()

        @pl.loop(0, _NCHUNK)
        def _(ci):
            buf = lax.rem(ci, _NBUF)
            prv = lax.rem(ci + _NBUF - 1, _NBUF)

            @pl.when(ci >= 1)
            def _():
                write(ci - 1, prv).wait()

            @pl.when(jnp.logical_and(ci >= 1, ci + _NBUF - 1 < _NCHUNK))
            def _():
                gather(ci + _NBUF - 1, prv).start()

            gather(ci, buf).wait()
            write(ci, buf).start()

        write(_NCHUNK - 1, lax.rem(_NCHUNK - 1, _NBUF)).wait()

    return k(idx_flat, table)


_T2_F = 2


def _unpack_relu_body(g_ref, o_ref):
    x = g_ref[...]
    xt = x.T
    for i in range(_T2_F):
        parts = [
            xt[a * _D:(a + 1) * _D, i * _T2_U:(i + 1) * _T2_U]
            for a in range(4)
        ]
        o_ref[i] = jnp.maximum(jnp.concatenate(parts, axis=1), 0.0)


def _unpack_relu(g128):
    return pl.pallas_call(
        _unpack_relu_body,
        grid=(_FIELDS // _T2_F,),
        in_specs=[pl.BlockSpec((_T2_F * _T2_U, 128), lambda f: (f, 0))],
        out_specs=pl.BlockSpec((_T2_F, _D, _BATCH), lambda f: (f, 0, 0)),
        out_shape=jax.ShapeDtypeStruct((_FIELDS, _D, _BATCH), jnp.float32),
        compiler_params=pltpu.CompilerParams(
            dimension_semantics=("parallel",),
        ),
    )(g128)


def kernel(indices, weight):
    wt = weight.T
    w2 = _pack_table(wt)
    table = w2.reshape(_T1_ROWS * 4, _D)

    it = indices.T.reshape(-1).astype(jnp.int32)
    vb = it // _T1_W
    u = it % _T1_W
    a = u // _T1_Q
    j = u % _T1_Q
    fidx = 4 * (vb * _T1_Q + j) + a

    g128 = _sc_gather(fidx, table)
    o_phys = _unpack_relu(g128)
    return o_phys.transpose(2, 0, 1)


# speedup vs baseline: 6.1621x; 1.0086x over previous
import functools

import jax
import jax.numpy as jnp
from jax import lax
from jax.experimental import pallas as pl
from jax.experimental.pallas import tpu as pltpu
from jax.experimental.pallas import tpu_sc as plsc

_DATASET = 1000000
_BATCH = 16384
_FIELDS = 26
_D = 32
_B = _BATCH * _FIELDS
_NC = 2
_NS = 16
_NW = _NC * _NS
_BPW = _B // _NW
_CH = 1024
_NCHUNK = _BPW // _CH
_NBUF = 3

_T1_W = 65536
_T1_Q = _T1_W // 4
_T1_STEPS = -(-_DATASET // _T1_W)
_T1_ROWS = _T1_STEPS * _T1_Q

_T2_U = _BATCH // 4


def _pack_table_body(wt_ref, ti_ref, out_ref, fidx_ref):
    x = wt_ref[...]
    z = jnp.concatenate(
        [x[:, a * _T1_Q:(a + 1) * _T1_Q] for a in range(4)], axis=0
    )
    out_ref[...] = z.T

    @pl.when(pl.program_id(0) == 0)
    def _():
        v = ti_ref[...]
        fidx_ref[...] = (
            jnp.bitwise_and(v, -_T1_W)
            + (jnp.bitwise_and(v, _T1_Q - 1) << 2)
            + jnp.bitwise_and(v >> 14, 3)
        )


def _pack_table(wt, ti):
    return pl.pallas_call(
        _pack_table_body,
        grid=(_T1_STEPS,),
        in_specs=[
            pl.BlockSpec((_D, _T1_W), lambda i: (0, i)),
            pl.BlockSpec((_FIELDS, _BATCH), lambda i: (0, 0)),
        ],
        out_specs=[
            pl.BlockSpec((_T1_Q, 128), lambda i: (i, 0)),
            pl.BlockSpec((_FIELDS, _BATCH), lambda i: (0, 0)),
        ],
        out_shape=[
            jax.ShapeDtypeStruct((_T1_ROWS, 128), jnp.float32),
            jax.ShapeDtypeStruct((_FIELDS, _BATCH), jnp.int32),
        ],
        compiler_params=pltpu.CompilerParams(
            dimension_semantics=("parallel",),
        ),
    )(wt, ti)


def _sc_gather(idx_flat, table):
    mesh = plsc.VectorSubcoreMesh(core_axis_name="c", subcore_axis_name="s")

    @functools.partial(
        pl.kernel,
        mesh=mesh,
        out_type=jax.ShapeDtypeStruct((_B // 4, 128), jnp.float32),
        compiler_params=pltpu.CompilerParams(use_tc_tiling_on_sc=False),
        scratch_types=[
            pltpu.VMEM((_BPW,), jnp.int32),
            pltpu.VMEM((_NBUF, _CH, _D), jnp.float32),
            pltpu.SemaphoreType.DMA((_NBUF,)),
            pltpu.SemaphoreType.DMA((_NBUF,)),
        ],
    )
    def k(idx_hbm, table_hbm, out_hbm, idx_v, rows_v, gsem, wsem):
        wid = lax.axis_index("s") * _NC + lax.axis_index("c")
        base = wid * _BPW
        pltpu.sync_copy(idx_hbm.at[pl.ds(base, _BPW)], idx_v)

        def gather(ci, buf):
            return pltpu.make_async_copy(
                table_hbm.at[idx_v.at[pl.ds(ci * _CH, _CH)]],
                rows_v.at[buf],
                gsem.at[buf],
            )

        def write(ci, buf):
            g0 = base + ci * _CH
            p = g0 // (_BATCH // 4)
            u0 = lax.rem(g0, _BATCH // 4)
            c0 = (p // 4) * (_BATCH // 4) + u0
            a = lax.rem(p, 4)
            return pltpu.make_async_copy(
                rows_v.at[buf],
                out_hbm.at[pl.ds(c0, _CH), pl.ds(a * _D, _D)],
                wsem.at[buf],
            )

        for k in range(_NBUF):
            gather(k, k).start()

        @pl.loop(0, _NCHUNK)
        def _(ci):
            buf = lax.rem(ci, _NBUF)
            prv = lax.rem(ci + _NBUF - 1, _NBUF)

            @pl.when(ci >= 1)
            def _():
                write(ci - 1, prv).wait()

            @pl.when(jnp.logical_and(ci >= 1, ci + _NBUF - 1 < _NCHUNK))
            def _():
                gather(ci + _NBUF - 1, prv).start()

            gather(ci, buf).wait()
            write(ci, buf).start()

        write(_NCHUNK - 1, lax.rem(_NCHUNK - 1, _NBUF)).wait()

    return k(idx_flat, table)


_T2_F = 2


def _unpack_relu_body(g_ref, o_ref):
    x = g_ref[...]
    xt = x.T
    for i in range(_T2_F):
        parts = [
            xt[a * _D:(a + 1) * _D, i * _T2_U:(i + 1) * _T2_U]
            for a in range(4)
        ]
        o_ref[i] = jnp.maximum(jnp.concatenate(parts, axis=1), 0.0)


def _unpack_relu(g128):
    return pl.pallas_call(
        _unpack_relu_body,
        grid=(_FIELDS // _T2_F,),
        in_specs=[pl.BlockSpec((_T2_F * _T2_U, 128), lambda f: (f, 0))],
        out_specs=pl.BlockSpec((_T2_F, _D, _BATCH), lambda f: (f, 0, 0)),
        out_shape=jax.ShapeDtypeStruct((_FIELDS, _D, _BATCH), jnp.float32),
        compiler_params=pltpu.CompilerParams(
            dimension_semantics=("parallel",),
        ),
    )(g128)


def kernel(indices, weight):
    wt = weight.T
    ti = indices.T.astype(jnp.int32)
    w2, fidx2 = _pack_table(wt, ti)
    table = w2.reshape(_T1_ROWS * 4, _D)
    fidx = fidx2.reshape(-1)

    g128 = _sc_gather(fidx, table)
    o_phys = _unpack_relu(g128)
    return o_phys.transpose(2, 0, 1)
